# speedup vs baseline: 1.3084x; 1.3084x over previous
import functools

import jax
import jax.numpy as jnp
from jax import lax
from jax.experimental import pallas as pl
from jax.experimental.pallas import tpu as pltpu
from jax.experimental.pallas import tpu_sc as plsc

_D = 128
_B = 5000
_NC = 2
_NS = 16
_NW = _NC * _NS
_ROWS = 160
_CHUNK = 80
_LAST_BASE = _B - _ROWS

_mesh = plsc.VectorSubcoreMesh(core_axis_name="c", subcore_axis_name="s")


@functools.partial(
    pl.kernel,
    mesh=_mesh,
    out_type=jax.ShapeDtypeStruct((_B, _D), jnp.float32),
    scratch_types=[
        pltpu.VMEM((_ROWS,), jnp.int32),
        pltpu.VMEM((_ROWS, _D), jnp.float32),
        pltpu.SemaphoreType.DMA,
    ],
)
def _gather_rows(idx_hbm, x_hbm, out_hbm, idx_v, rows_v, sem):
    wid = lax.axis_index("s") * _NC + lax.axis_index("c")
    base = pl.multiple_of(lax.min(wid * _ROWS, _LAST_BASE), 8)
    pltpu.sync_copy(idx_hbm.at[pl.ds(base, _ROWS)], idx_v)
    copies = []
    for j in range(_ROWS // _CHUNK):
        copies.append(
            pltpu.async_copy(
                x_hbm.at[idx_v.at[pl.ds(j * _CHUNK, _CHUNK)]],
                rows_v.at[pl.ds(j * _CHUNK, _CHUNK)],
                sem,
            )
        )
    for cp in copies:
        cp.wait()
    pltpu.sync_copy(rows_v, out_hbm.at[pl.ds(base, _ROWS)])


def kernel(node_indices, x, edge_index, edge_type, edge_attr):
    del edge_index, edge_type, edge_attr
    return _gather_rows(node_indices, x)
